# baseline (speedup 1.0000x reference)
import jax
import jax.numpy as jnp
from jax import lax
from jax.experimental import pallas as pl
from jax.experimental.pallas import tpu as pltpu
from jax.experimental.pallas import tpu_sc as plsc

_B, _N, _Q = 16, 2048, 128
_SPAD = 2 * _Q
_STRIDE = 133
_NPTS = _N // 2
_UNROLL = 2


def _tc_prep_body(srow_ref, scol_ref, spad_ref, ranks_ref):
    srow = srow_ref[...]
    scol = scol_ref[...]
    krow = lax.broadcasted_iota(jnp.int32, (_Q, _Q), 0)
    jcol = lax.broadcasted_iota(jnp.int32, (_Q, _Q), 1)
    m = (scol < srow) | ((scol == srow) & (krow < jcol))
    ranks_f = jnp.sum(m.astype(jnp.float32), axis=0, keepdims=True)
    ranks_ref[...] = ranks_f.astype(jnp.int32)
    mt = (srow < scol) | ((srow == scol) & (jcol < krow))
    rank_col = jnp.sum(mt.astype(jnp.float32), axis=1, keepdims=False)
    oh = (rank_col[:, None] == jcol.astype(jnp.float32)).astype(jnp.float32)
    spad_ref[:, :_Q] = lax.dot_general(
        srow, oh, (((1,), (0,)), ((), ())), preferred_element_type=jnp.float32)
    spad_ref[:, _Q:] = jnp.full((1, _Q), 2.0, jnp.float32)


_tc_prep = pl.pallas_call(
    _tc_prep_body,
    out_shape=[
        jax.ShapeDtypeStruct((1, _SPAD), jnp.float32),
        jax.ShapeDtypeStruct((1, _Q), jnp.int32),
    ],
)


def _sc_tile(b, h, xs_hbm, ys_hbm, spad_hbm, ranks_hbm, out_hbm,
             xs_v, ys_v, sp_v, rk_v, da_v, db_v, ab_v, f_v):
    pltpu.sync_copy(xs_hbm.at[b, h], xs_v)
    pltpu.sync_copy(ys_hbm.at[b, h], ys_v)
    pltpu.sync_copy(spad_hbm, sp_v)
    pltpu.sync_copy(ranks_hbm, rk_v)

    zero = jnp.zeros((16,), jnp.float32)
    for i in range(_STRIDE):
        da_v[pl.ds(16 * i, 16)] = zero
        db_v[pl.ds(16 * i, 16)] = zero

    lanebase = lax.iota(jnp.int32, 16) * _STRIDE
    ones = jnp.ones((16,), jnp.float32)
    neg2 = jnp.full((16,), -2.0, jnp.float32)

    def count_le(v):
        p = jnp.full((16,), _Q - 1, jnp.int32)
        for d in (128, 64, 32, 16, 8, 4, 2):
            cle = plsc.load_gather(sp_v, [p]) <= v
            d2 = d // 2
            p = p + jnp.where(cle, d2, d2 - d).astype(jnp.int32)
        return p + (plsc.load_gather(sp_v, [p]) <= v).astype(jnp.int32)

    def body(i, carry):
        for u in range(_UNROLL):
            g = i * _UNROLL + u
            xv = xs_v[pl.ds(g * 16, 16)]
            yv = ys_v[pl.ds(g * 16, 16)]
            m2 = xv + yv
            mv = 0.5 * m2
            cx = lanebase + count_le(xv)
            cm = lanebase + count_le(mv)
            cy = lanebase + count_le(yv)
            plsc.addupdate_scatter(da_v, [cx], ones)
            plsc.addupdate_scatter(da_v, [cm], neg2)
            plsc.addupdate_scatter(da_v, [cy], ones)
            plsc.addupdate_scatter(db_v, [cx], 0.0 - xv)
            plsc.addupdate_scatter(db_v, [cm], m2)
            plsc.addupdate_scatter(db_v, [cy], 0.0 - yv)
        return carry

    lax.fori_loop(0, _NPTS // (16 * _UNROLL), body, 0)

    carry_a = zero
    carry_b = zero
    for i in range(_Q // 16):
        va = zero
        vb = zero
        for l in range(16):
            va = va + da_v[pl.ds(l * _STRIDE + 16 * i, 16)]
            vb = vb + db_v[pl.ds(l * _STRIDE + 16 * i, 16)]
        ca = plsc.cumsum(va) + carry_a
        cb = plsc.cumsum(vb) + carry_b
        carry_a = jnp.full((16,), 0.0, jnp.float32) + ca[15]
        carry_b = jnp.full((16,), 0.0, jnp.float32) + cb[15]
        ab_v[pl.ds(16 * i, 16)] = ca
        ab_v[pl.ds(_Q + 16 * i, 16)] = cb

    for i in range(_Q // 16):
        rv = rk_v[pl.ds(16 * i, 16)]
        av = plsc.load_gather(ab_v, [rv])
        bv = plsc.load_gather(ab_v, [rv + _Q])
        tv = plsc.load_gather(sp_v, [rv])
        f_v[pl.ds(16 * i, 16)] = tv * av + bv
    pltpu.sync_copy(f_v, out_hbm.at[b, h])


def _sc_pool_body(xs_hbm, ys_hbm, spad_hbm, ranks_hbm, out_hbm,
                  xs_v, ys_v, sp_v, rk_v, da_v, db_v, ab_v, f_v):
    c = lax.axis_index("c")
    s = lax.axis_index("s")
    _sc_tile(s, c, xs_hbm, ys_hbm, spad_hbm, ranks_hbm, out_hbm,
             xs_v, ys_v, sp_v, rk_v, da_v, db_v, ab_v, f_v)


_SC_SCRATCH = [
    pltpu.VMEM((_NPTS,), jnp.float32),
    pltpu.VMEM((_NPTS,), jnp.float32),
    pltpu.VMEM((_SPAD,), jnp.float32),
    pltpu.VMEM((_Q,), jnp.int32),
    pltpu.VMEM((16 * _STRIDE,), jnp.float32),
    pltpu.VMEM((16 * _STRIDE,), jnp.float32),
    pltpu.VMEM((2 * _Q,), jnp.float32),
    pltpu.VMEM((_Q,), jnp.float32),
]

_SC_POOL_CACHE = []


def _sc_pool(*args):
    if not _SC_POOL_CACHE:
        _SC_POOL_CACHE.append(pl.kernel(
            _sc_pool_body,
            out_type=jax.ShapeDtypeStruct((_B, 2, _Q), jnp.float32),
            mesh=plsc.VectorSubcoreMesh(core_axis_name="c",
                                        subcore_axis_name="s"),
            compiler_params=pltpu.CompilerParams(needs_layout_passes=False),
            scratch_types=_SC_SCRATCH,
        ))
    return _SC_POOL_CACHE[0](*args)


def _tc_rho_body(f0_ref, f1_ref, w_ref, b_ref, out_ref):
    pooled = f0_ref[...] + f1_ref[...]
    acc = lax.dot_general(
        pooled, w_ref[...], (((1,), (0,)), ((), ())),
        preferred_element_type=jnp.float32,
    )
    out_ref[...] = jnp.maximum(acc + b_ref[...], 0.0)


_tc_rho = pl.pallas_call(
    _tc_rho_body,
    out_shape=jax.ShapeDtypeStruct((_B, _Q), jnp.float32),
)


def kernel(diagram, samples, rho_w, rho_b):
    xs = diagram[:, :, 0].reshape(_B, 2, _NPTS)
    ys = diagram[:, :, 1].reshape(_B, 2, _NPTS)
    spad, ranks = _tc_prep(samples.reshape(1, _Q), samples.reshape(_Q, 1))
    f = _sc_pool(xs, ys, spad.reshape(_SPAD), ranks.reshape(_Q))
    return _tc_rho(f[:, 0], f[:, 1], rho_w.T, rho_b.reshape(1, _Q))
